# baseline (speedup 1.0000x reference)
import functools

import jax
import jax.numpy as jnp
from jax import lax
from jax.experimental import pallas as pl
from jax.experimental.pallas import tpu as pltpu
from jax.experimental.pallas import tpu_sc as plsc

N_ROWS = 16384
N_ATOMS = 200
NUM_WORKERS = 32
COLS_PER_W = N_ROWS // NUM_WORKERS
CHUNK_COLS = 256
NUM_CHUNKS = COLS_PER_W // CHUNK_COLS
LANES = 16
GROUPS = CHUNK_COLS // LANES


def _sc_body(spT_hbm, energies_hbm, se_hbm, out_spT_hbm, out_hbm,
             sp0_v, sp1_v, se_v, tbl2_v, tbl_v, en_v, row_v,
             sem_in0, sem_in1, sem_out0, sem_out1):
    wid = lax.axis_index("s") * 2 + lax.axis_index("c")
    base_col = wid * COLS_PER_W

    in0 = pltpu.async_copy(
        spT_hbm.at[:, pl.ds(base_col, CHUNK_COLS)], sp0_v, sem_in0)
    in1 = pltpu.async_copy(
        spT_hbm.at[:, pl.ds(base_col + CHUNK_COLS, CHUNK_COLS)], sp1_v,
        sem_in1)

    pltpu.sync_copy(se_hbm, se_v)
    lane = lax.iota(jnp.int32, LANES)
    for j in range(4):
        lo = lane & 7
        hi = (lane >> 3) + 2 * j
        tbl2_v[pl.ds(16 * j, LANES)] = (plsc.load_gather(se_v, [lo])
                                        + plsc.load_gather(se_v, [hi]))

    @plsc.parallel_loop(0, 256, unroll=4)
    def tbl_body(v):
        lo = (v & 3) * 16 + lane
        hi = jnp.full((LANES,), v >> 2, jnp.int32)
        tbl_v[pl.ds(v * 16, LANES)] = (plsc.load_gather(tbl2_v, [lo])
                                       + plsc.load_gather(tbl2_v, [hi]))

    pltpu.sync_copy(energies_hbm.at[pl.ds(base_col, COLS_PER_W)], en_v)

    for c, (sp_v, cin, sem_out) in enumerate(
            ((sp0_v, in0, sem_out0), (sp1_v, in1, sem_out1))):
        col0 = base_col + c * CHUNK_COLS
        cin.wait()
        wb = pltpu.async_copy(sp_v, out_spT_hbm.at[:, pl.ds(col0, CHUNK_COLS)],
                              sem_out)

        if c == 1:
            wb.wait()
            continue

        @plsc.parallel_loop(0, GROUPS, unroll=1)
        def group_body(g):
            lb = g * LANES
            zero = jnp.zeros((LANES,), jnp.float32)

            @plsc.parallel_loop(0, N_ATOMS // 8, unroll=5,
                                carry=(zero, zero))
            def quad_pair(qi, accs):
                acc0, acc1 = accs
                a = 8 * qi
                for j, _ in enumerate(accs):
                    b = a + 4 * j
                    s0 = sp_v[b, pl.ds(lb, LANES)]
                    s1 = sp_v[b + 1, pl.ds(lb, LANES)]
                    s2 = sp_v[b + 2, pl.ds(lb, LANES)]
                    s3 = sp_v[b + 3, pl.ds(lb, LANES)]
                    idx = s0 + s1 * 8 + s2 * 64 + s3 * 512
                    gathered = plsc.load_gather(tbl_v, [idx])
                    if j == 0:
                        acc0 = acc0 + gathered
                    else:
                        acc1 = acc1 + gathered
                return (acc0, acc1)

            acc0, acc1 = quad_pair
            ev = en_v[pl.ds(c * CHUNK_COLS + lb, LANES)]
            row_v[pl.ds(lb, LANES)] = acc0 + acc1 + ev

        pltpu.sync_copy(row_v, out_hbm.at[pl.ds(col0, CHUNK_COLS)])
        wb.wait()


@jax.jit
def _run(spT, energies, se16):
    mesh = plsc.VectorSubcoreMesh(core_axis_name="c", subcore_axis_name="s")
    return pl.kernel(
        _sc_body,
        mesh=mesh,
        compiler_params=pltpu.CompilerParams(needs_layout_passes=False),
        out_type=(jax.ShapeDtypeStruct((N_ATOMS, N_ROWS), jnp.int32),
                  jax.ShapeDtypeStruct((N_ROWS,), jnp.float32)),
        scratch_types=[
            pltpu.VMEM((N_ATOMS, CHUNK_COLS), jnp.int32),
            pltpu.VMEM((N_ATOMS, CHUNK_COLS), jnp.int32),
            pltpu.VMEM((LANES,), jnp.float32),
            pltpu.VMEM((64,), jnp.float32),
            pltpu.VMEM((4096,), jnp.float32),
            pltpu.VMEM((COLS_PER_W,), jnp.float32),
            pltpu.VMEM((CHUNK_COLS,), jnp.float32),
            pltpu.SemaphoreType.DMA,
            pltpu.SemaphoreType.DMA,
            pltpu.SemaphoreType.DMA,
            pltpu.SemaphoreType.DMA,
        ],
    )(spT, energies, se16)


TC_BLOCK = CHUNK_COLS


def _tc_body(sp_ref, en_ref, se_ref, out_ref):
    sp = sp_ref[...]
    b0 = (sp & 1) != 0
    b1 = (sp & 2) != 0
    b2 = (sp & 4) != 0
    t01 = jnp.where(b0, se_ref[1], se_ref[0])
    t23 = jnp.where(b0, se_ref[3], se_ref[2])
    t45 = jnp.where(b0, se_ref[5], se_ref[4])
    t67 = jnp.where(b0, se_ref[7], se_ref[6])
    m0 = jnp.where(b1, t23, t01)
    m1 = jnp.where(b1, t67, t45)
    val = jnp.where(b2, m1, m0)
    out_ref[...] = val.sum(axis=0) + en_ref[...]


@jax.jit
def _run_tc(spT, energies, se):
    return pl.pallas_call(
        _tc_body,
        grid=(NUM_WORKERS,),
        in_specs=[
            pl.BlockSpec((N_ATOMS, TC_BLOCK), lambda j: (0, 2 * j + 1)),
            pl.BlockSpec((TC_BLOCK,), lambda j: (2 * j + 1,)),
            pl.BlockSpec(memory_space=pltpu.SMEM),
        ],
        out_specs=pl.BlockSpec((TC_BLOCK,), lambda j: (2 * j + 1,)),
        out_shape=jax.ShapeDtypeStruct((N_ROWS,), jnp.float32),
    )(spT, energies, se)


def kernel(species, energies, self_energies):
    se = self_energies.astype(jnp.float32)
    se16 = jnp.pad(se, (0, 8))
    spT = species.T
    out_spT, shifted_sc = _run(spT, energies, se16)
    shifted_tc = _run_tc(spT, energies, se)
    blk = (lax.iota(jnp.int32, N_ROWS) // CHUNK_COLS) % 2
    shifted = jnp.where(blk == 1, shifted_tc, shifted_sc)
    return (out_spT.T, shifted)


# speedup vs baseline: 1.2804x; 1.2804x over previous
import functools

import jax
import jax.numpy as jnp
from jax import lax
from jax.experimental import pallas as pl
from jax.experimental.pallas import tpu as pltpu
from jax.experimental.pallas import tpu_sc as plsc

N_ROWS = 16384
N_ATOMS = 200
NUM_WORKERS = 32
COLS_PER_W = N_ROWS // NUM_WORKERS
CHUNK_COLS = 256
NUM_CHUNKS = COLS_PER_W // CHUNK_COLS
LANES = 16
GROUPS = CHUNK_COLS // LANES


def _sc_body(spT_hbm, energies_hbm, se_hbm, out_spT_hbm, out_hbm,
             sp0_v, sp1_v, se_v, tbl2_v, tbl_v, en_v, row_v,
             sem_in0, sem_in1, sem_out0, sem_out1):
    wid = lax.axis_index("s") * 2 + lax.axis_index("c")
    base_col = wid * COLS_PER_W

    in0 = pltpu.async_copy(
        spT_hbm.at[:, pl.ds(base_col, CHUNK_COLS)], sp0_v, sem_in0)
    in1 = pltpu.async_copy(
        spT_hbm.at[:, pl.ds(base_col + CHUNK_COLS, CHUNK_COLS)], sp1_v,
        sem_in1)

    pltpu.sync_copy(se_hbm, se_v)
    lane = lax.iota(jnp.int32, LANES)
    for j in range(4):
        lo = lane & 7
        hi = (lane >> 3) + 2 * j
        tbl2_v[pl.ds(16 * j, LANES)] = (plsc.load_gather(se_v, [lo])
                                        + plsc.load_gather(se_v, [hi]))

    @plsc.parallel_loop(0, 256, unroll=4)
    def tbl_body(v):
        lo = (v & 3) * 16 + lane
        hi = jnp.full((LANES,), v >> 2, jnp.int32)
        tbl_v[pl.ds(v * 16, LANES)] = (plsc.load_gather(tbl2_v, [lo])
                                       + plsc.load_gather(tbl2_v, [hi]))

    pltpu.sync_copy(energies_hbm.at[pl.ds(base_col, COLS_PER_W)], en_v)

    for c, (sp_v, cin, sem_out) in enumerate(
            ((sp0_v, in0, sem_out0), (sp1_v, in1, sem_out1))):
        col0 = base_col + c * CHUNK_COLS
        cin.wait()
        wb = pltpu.async_copy(sp_v, out_spT_hbm.at[:, pl.ds(col0, CHUNK_COLS)],
                              sem_out)

        @plsc.parallel_loop(0, GROUPS, unroll=1)
        def group_body(g):
            lb = g * LANES
            zero = jnp.zeros((LANES,), jnp.float32)

            @plsc.parallel_loop(0, N_ATOMS // 8, unroll=5,
                                carry=(zero, zero))
            def quad_pair(qi, accs):
                acc0, acc1 = accs
                a = 8 * qi
                for j, _ in enumerate(accs):
                    b = a + 4 * j
                    s0 = sp_v[b, pl.ds(lb, LANES)]
                    s1 = sp_v[b + 1, pl.ds(lb, LANES)]
                    s2 = sp_v[b + 2, pl.ds(lb, LANES)]
                    s3 = sp_v[b + 3, pl.ds(lb, LANES)]
                    idx = s0 + s1 * 8 + s2 * 64 + s3 * 512
                    gathered = plsc.load_gather(tbl_v, [idx])
                    if j == 0:
                        acc0 = acc0 + gathered
                    else:
                        acc1 = acc1 + gathered
                return (acc0, acc1)

            acc0, acc1 = quad_pair
            ev = en_v[pl.ds(c * CHUNK_COLS + lb, LANES)]
            row_v[pl.ds(lb, LANES)] = acc0 + acc1 + ev

        pltpu.sync_copy(row_v, out_hbm.at[pl.ds(col0, CHUNK_COLS)])
        wb.wait()


@jax.jit
def _run(spT, energies, se8):
    mesh = plsc.VectorSubcoreMesh(core_axis_name="c", subcore_axis_name="s")
    return pl.kernel(
        _sc_body,
        mesh=mesh,
        compiler_params=pltpu.CompilerParams(needs_layout_passes=False),
        out_type=(jax.ShapeDtypeStruct((N_ATOMS, N_ROWS), jnp.int32),
                  jax.ShapeDtypeStruct((N_ROWS,), jnp.float32)),
        scratch_types=[
            pltpu.VMEM((N_ATOMS, CHUNK_COLS), jnp.int32),
            pltpu.VMEM((N_ATOMS, CHUNK_COLS), jnp.int32),
            pltpu.VMEM((8,), jnp.float32),
            pltpu.VMEM((64,), jnp.float32),
            pltpu.VMEM((4096,), jnp.float32),
            pltpu.VMEM((COLS_PER_W,), jnp.float32),
            pltpu.VMEM((CHUNK_COLS,), jnp.float32),
            pltpu.SemaphoreType.DMA,
            pltpu.SemaphoreType.DMA,
            pltpu.SemaphoreType.DMA,
            pltpu.SemaphoreType.DMA,
        ],
    )(spT, energies, se8)


def kernel(species, energies, self_energies):
    out_spT, shifted = _run(species.T, energies,
                            self_energies.astype(jnp.float32))
    return (out_spT.T, shifted)
